# baseline (speedup 1.0000x reference)
import functools

import jax
import jax.numpy as jnp
from jax import lax
from jax.experimental import pallas as pl
from jax.experimental.pallas import tpu as pltpu
from jax.experimental.pallas import tpu_sc as plsc

HASH_BITS = 20
HASH_SIZE = 2 ** HASH_BITS
DIM = 64
N = 16384
PRIME_X = 73856093
PRIME_Y = 19349663

_INFO = plsc.get_sparse_core_info()
_NC = _INFO.num_cores
_NS = _INFO.num_subcores
_NW = _NC * _NS
_BPW = N // _NW
_PIPE = 32


def _sc_body(pos_hbm, table_hbm, out_hbm, pos_v, idx_s, sem):
    c = lax.axis_index("c")
    s = lax.axis_index("s")
    wid = s * _NC + c
    base = wid * _BPW

    pltpu.sync_copy(pos_hbm.at[pl.ds(2 * base, 2 * _BPW)], pos_v)

    lane = lax.iota(jnp.int32, 16)

    def hash_of(p):
        t = p.astype(jnp.int32)
        return t - (t.astype(jnp.float32) > p).astype(jnp.int32)

    for i in range(_BPW // 16):
        gx = lane * 2 + (2 * 16 * i)
        px = plsc.load_gather(pos_v, [gx])
        py = plsc.load_gather(pos_v, [gx + 1])
        h = ((hash_of(px) * PRIME_X) ^ (hash_of(py) * PRIME_Y)) \
            & (HASH_SIZE - 1)
        for l in range(16):
            idx_s[i * 16 + l] = h[l]

    def row_dma(i):
        pltpu.async_copy(
            table_hbm.at[pl.ds(idx_s[i], 1)],
            out_hbm.at[pl.ds(base + i, 1)],
            sem,
        )

    def drain_one():
        pltpu.make_async_copy(
            table_hbm.at[pl.ds(0, 1)],
            out_hbm.at[pl.ds(base, 1)],
            sem,
        ).wait()

    def _step(_, i):
        row_dma(i)
        drain_one()
        return i + 1

    for i in range(_PIPE):
        row_dma(jnp.int32(i))
    lax.fori_loop(0, _BPW - _PIPE, _step, jnp.int32(_PIPE), unroll=4)
    for _ in range(_PIPE):
        drain_one()


@jax.jit
def _hash_grid_lookup(pos_flat, table):
    mesh = plsc.VectorSubcoreMesh(core_axis_name="c", subcore_axis_name="s")
    k = functools.partial(
        pl.kernel,
        mesh=mesh,
        compiler_params=pltpu.CompilerParams(needs_layout_passes=False),
        out_type=jax.ShapeDtypeStruct((N, DIM), jnp.float32),
        scratch_types=[
            pltpu.VMEM((2 * _BPW,), jnp.float32),
            pltpu.SMEM((_BPW,), jnp.int32),
            pltpu.SemaphoreType.DMA,
        ],
    )(_sc_body)
    return k(pos_flat, table)


def kernel(positions, table):
    pos_flat = positions.reshape(2 * N)
    return _hash_grid_lookup(pos_flat, table)


# speedup vs baseline: 1.6229x; 1.6229x over previous
import functools

import jax
import jax.numpy as jnp
from jax import lax
from jax.experimental import pallas as pl
from jax.experimental.pallas import tpu as pltpu
from jax.experimental.pallas import tpu_sc as plsc

HASH_BITS = 20
HASH_SIZE = 2 ** HASH_BITS
DIM = 64
N = 16384
PRIME_X = 73856093
PRIME_Y = 19349663

_INFO = plsc.get_sparse_core_info()
_NC = _INFO.num_cores
_NS = _INFO.num_subcores
_NW = _NC * _NS
_BPW = N // _NW
_PIPE = 64


def _sc_body(pos_hbm, table_hbm, out_hbm, pos_v, rows_v, idx_s, sem):
    c = lax.axis_index("c")
    s = lax.axis_index("s")
    wid = s * _NC + c
    base = wid * _BPW

    pltpu.sync_copy(pos_hbm.at[pl.ds(2 * base, 2 * _BPW)], pos_v)

    lane = lax.iota(jnp.int32, 16)

    def hash_of(p):
        t = p.astype(jnp.int32)
        return t - (t.astype(jnp.float32) > p).astype(jnp.int32)

    for i in range(_BPW // 16):
        gx = lane * 2 + (2 * 16 * i)
        px = plsc.load_gather(pos_v, [gx])
        py = plsc.load_gather(pos_v, [gx + 1])
        h = ((hash_of(px) * PRIME_X) ^ (hash_of(py) * PRIME_Y)) \
            & (HASH_SIZE - 1)
        for l in range(16):
            idx_s[i * 16 + l] = h[l]

    def row_dma(i):
        pltpu.async_copy(
            table_hbm.at[pl.ds(idx_s[i], 1)],
            rows_v.at[pl.ds(i, 1)],
            sem,
        )

    def drain_one():
        pltpu.make_async_copy(
            table_hbm.at[pl.ds(0, 1)],
            rows_v.at[pl.ds(0, 1)],
            sem,
        ).wait()

    def _step(_, i):
        row_dma(i)
        drain_one()
        return i + 1

    for i in range(_PIPE):
        row_dma(jnp.int32(i))
    lax.fori_loop(0, _BPW - _PIPE, _step, jnp.int32(_PIPE), unroll=8)
    for _ in range(_PIPE):
        drain_one()

    pltpu.sync_copy(rows_v, out_hbm.at[pl.ds(base, _BPW)])


@jax.jit
def _hash_grid_lookup(pos_flat, table):
    mesh = plsc.VectorSubcoreMesh(core_axis_name="c", subcore_axis_name="s")
    k = functools.partial(
        pl.kernel,
        mesh=mesh,
        compiler_params=pltpu.CompilerParams(needs_layout_passes=False),
        out_type=jax.ShapeDtypeStruct((N, DIM), jnp.float32),
        scratch_types=[
            pltpu.VMEM((2 * _BPW,), jnp.float32),
            pltpu.VMEM((_BPW, DIM), jnp.float32),
            pltpu.SMEM((_BPW,), jnp.int32),
            pltpu.SemaphoreType.DMA,
        ],
    )(_sc_body)
    return k(pos_flat, table)


def kernel(positions, table):
    pos_flat = positions.reshape(2 * N)
    return _hash_grid_lookup(pos_flat, table)


# speedup vs baseline: 6.6400x; 4.0915x over previous
import functools

import jax
import jax.numpy as jnp
from jax import lax
from jax.experimental import pallas as pl
from jax.experimental.pallas import tpu as pltpu
from jax.experimental.pallas import tpu_sc as plsc

HASH_BITS = 20
HASH_SIZE = 2 ** HASH_BITS
DIM = 64
N = 16384
PRIME_X = 73856093
PRIME_Y = 19349663

_INFO = plsc.get_sparse_core_info()
_NC = _INFO.num_cores
_NS = _INFO.num_subcores
_NW = _NC * _NS
_BPW = N // _NW
_NSTREAM = _BPW * DIM // 128
_PIPE = 16


def _sc_body(pos_hbm, tab_hbm, out_t_hbm, pos_v, a_v, idx_v, out_t_v, sem):
    c = lax.axis_index("c")
    s = lax.axis_index("s")
    wid = s * _NC + c
    base = wid * _BPW

    pltpu.sync_copy(pos_hbm.at[pl.ds(2 * base, 2 * _BPW)], pos_v)

    lane = lax.iota(jnp.int32, 16)

    def hash_of(p):
        t = p.astype(jnp.int32)
        return t - (t.astype(jnp.float32) > p).astype(jnp.int32)

    for i in range(_BPW // 16):
        gx = lane * 2 + (2 * 16 * i)
        px = plsc.load_gather(pos_v, [gx])
        py = plsc.load_gather(pos_v, [gx + 1])
        h = ((hash_of(px) * PRIME_X) ^ (hash_of(py) * PRIME_Y)) \
            & (HASH_SIZE - 1)
        a_v[pl.ds(i * 16, 16)] = ((h >> 7) << 10) + (h & 127)

    for d in range(DIM):
        c_d = (d >> 3) * (2 ** 23) + (d & 7) * 128
        for jj in range(_BPW // 128):
            for g in range(8):
                idx_v[d * (_BPW // 128) + jj, pl.ds(g * 16, 16)] = (
                    a_v[pl.ds(jj * 128 + g * 16, 16)] + c_d
                )

    def fire(j):
        return pltpu.async_copy(
            tab_hbm.at[idx_v.at[jnp.int32(j)]],
            out_t_v.at[jnp.int32(j // (_BPW // 128)),
                       pl.ds((j % (_BPW // 128)) * 128, 128)],
            sem,
        )

    def drain_one():
        pltpu.make_async_copy(
            tab_hbm.at[pl.ds(0, 128)],
            out_t_v.at[jnp.int32(0), pl.ds(0, 128)],
            sem,
        ).wait()

    for j in range(_NSTREAM):
        fire(j)
        if j >= _PIPE:
            drain_one()
    for _ in range(_PIPE):
        drain_one()

    pltpu.sync_copy(out_t_v, out_t_hbm.at[:, pl.ds(base, _BPW)])


@jax.jit
def _hash_grid_lookup(pos_flat, tab_flat):
    mesh = plsc.VectorSubcoreMesh(core_axis_name="c", subcore_axis_name="s")
    k = functools.partial(
        pl.kernel,
        mesh=mesh,
        compiler_params=pltpu.CompilerParams(
            needs_layout_passes=False, use_tc_tiling_on_sc=False
        ),
        out_type=jax.ShapeDtypeStruct((DIM, N), jnp.float32),
        scratch_types=[
            pltpu.VMEM((2 * _BPW,), jnp.float32),
            pltpu.VMEM((_BPW,), jnp.int32),
            pltpu.VMEM((_NSTREAM, 128), jnp.int32),
            pltpu.VMEM((DIM, _BPW), jnp.float32),
            pltpu.SemaphoreType.DMA,
        ],
    )(_sc_body)
    return k(pos_flat, tab_flat)


def kernel(positions, table):
    pos_flat = positions.reshape(2 * N)
    tab_flat = (
        table.reshape(HASH_SIZE // 128, 128, DIM // 8, 8)
        .transpose(2, 0, 3, 1)
        .reshape(HASH_SIZE * DIM)
    )
    out_t = _hash_grid_lookup(pos_flat, tab_flat)
    return out_t.T


# speedup vs baseline: 7.8972x; 1.1893x over previous
import functools

import jax
import jax.numpy as jnp
from jax import lax
from jax.experimental import pallas as pl
from jax.experimental.pallas import tpu as pltpu
from jax.experimental.pallas import tpu_sc as plsc

HASH_BITS = 20
HASH_SIZE = 2 ** HASH_BITS
DIM = 64
N = 16384
PRIME_X = 73856093
PRIME_Y = 19349663

_INFO = plsc.get_sparse_core_info()
_NC = _INFO.num_cores
_NS = _INFO.num_subcores
_NW = _NC * _NS
_BPW = N // _NW
_NSTREAM = _BPW * DIM // 128
_PIPE = 32


def _sc_body(pos_hbm, tab_hbm, out4_hbm, pos_v, a_v, idx_v, out_t_v,
             sem, osem):
    c = lax.axis_index("c")
    s = lax.axis_index("s")
    wid = s * _NC + c
    base = wid * _BPW

    pltpu.sync_copy(pos_hbm.at[pl.ds(2 * base, 2 * _BPW)], pos_v)

    def hash_of(p):
        t = p.astype(jnp.int32)
        return t - (t.astype(jnp.float32) > p).astype(jnp.int32)

    for i in range(_BPW // 16):
        o = (i // 8) * 256 + (i % 8) * 16
        px = pos_v[pl.ds(o, 16)]
        py = pos_v[pl.ds(o + 128, 16)]
        h = ((hash_of(px) * PRIME_X) ^ (hash_of(py) * PRIME_Y)) \
            & (HASH_SIZE - 1)
        a_v[pl.ds(i * 16, 16)] = ((h >> 7) << 10) + (h & 127)

    for d in range(DIM):
        c_d = (d >> 3) * (2 ** 23) + (d & 7) * 128
        for jj in range(_BPW // 128):
            for g in range(8):
                idx_v[d * (_BPW // 128) + jj, pl.ds(g * 16, 16)] = (
                    a_v[pl.ds(jj * 128 + g * 16, 16)] + c_d
                )

    def fire(j):
        return pltpu.async_copy(
            tab_hbm.at[idx_v.at[jnp.int32(j)]],
            out_t_v.at[jnp.int32(j // (_BPW // 128)),
                       pl.ds((j % (_BPW // 128)) * 128, 128)],
            sem,
        )

    def drain_one():
        pltpu.make_async_copy(
            tab_hbm.at[pl.ds(0, 128)],
            out_t_v.at[jnp.int32(0), pl.ds(0, 128)],
            sem,
        ).wait()

    for j in range(_NSTREAM):
        fire(j)
        if j >= _PIPE:
            drain_one()
    for _ in range(_PIPE):
        drain_one()

    for d_hi in range(DIM // 8):
        for gi in range(_BPW // 128):
            pltpu.async_copy(
                out_t_v.at[pl.ds(d_hi * 8, 8), pl.ds(gi * 128, 128)],
                out4_hbm.at[jnp.int32(d_hi), jnp.int32(wid * 4 + gi)],
                osem,
            )
    for d_hi in range(DIM // 8):
        for gi in range(_BPW // 128):
            pltpu.make_async_copy(
                out_t_v.at[pl.ds(0, 8), pl.ds(0, 128)],
                out4_hbm.at[jnp.int32(0), jnp.int32(0)],
                osem,
            ).wait()


@jax.jit
def _hash_grid_lookup(pos_flat, tab_flat):
    mesh = plsc.VectorSubcoreMesh(core_axis_name="c", subcore_axis_name="s")
    k = functools.partial(
        pl.kernel,
        mesh=mesh,
        compiler_params=pltpu.CompilerParams(
            needs_layout_passes=False, use_tc_tiling_on_sc=False
        ),
        out_type=jax.ShapeDtypeStruct((DIM // 8, N // 128, 8, 128),
                                      jnp.float32),
        scratch_types=[
            pltpu.VMEM((2 * _BPW,), jnp.float32),
            pltpu.VMEM((_BPW,), jnp.int32),
            pltpu.VMEM((_NSTREAM, 128), jnp.int32),
            pltpu.VMEM((DIM, _BPW), jnp.float32),
            pltpu.SemaphoreType.DMA,
            pltpu.SemaphoreType.DMA,
        ],
    )(_sc_body)
    return k(pos_flat, tab_flat)


def kernel(positions, table):
    pos_flat = (
        positions.reshape(N // 128, 128, 2).transpose(0, 2, 1).reshape(2 * N)
    )
    tab_flat = (
        table.reshape(HASH_SIZE // 128, 128, DIM // 8, 8)
        .transpose(2, 0, 3, 1)
        .reshape(HASH_SIZE * DIM)
    )
    out4 = _hash_grid_lookup(pos_flat, tab_flat)
    return out4.transpose(1, 3, 0, 2).reshape(N, DIM)


# speedup vs baseline: 9.2427x; 1.1704x over previous
import functools

import jax
import jax.numpy as jnp
from jax import lax
from jax.experimental import pallas as pl
from jax.experimental.pallas import tpu as pltpu
from jax.experimental.pallas import tpu_sc as plsc

HASH_BITS = 20
HASH_SIZE = 2 ** HASH_BITS
DIM = 64
N = 16384
PRIME_X = 73856093
PRIME_Y = 19349663

_INFO = plsc.get_sparse_core_info()
_NC = _INFO.num_cores
_NS = _INFO.num_subcores
_NW = _NC * _NS
_BPW = N // _NW
_NSTREAM = _BPW * DIM // 128
_PIPE = 32


def _sc_body(pos_hbm, tab_hbm, out4_hbm, pos_v, a_v, idx_v, out_t_v,
             g0, g1, g2, g3, g4, g5, g6, g7, osem):
    gsems = (g0, g1, g2, g3, g4, g5, g6, g7)
    c = lax.axis_index("c")
    s = lax.axis_index("s")
    wid = s * _NC + c
    base = wid * _BPW

    pltpu.sync_copy(pos_hbm.at[pl.ds(2 * base, 2 * _BPW)], pos_v)

    def hash_of(p):
        t = p.astype(jnp.int32)
        return t - (t.astype(jnp.float32) > p).astype(jnp.int32)

    for i in range(_BPW // 16):
        o = (i // 8) * 256 + (i % 8) * 16
        px = pos_v[pl.ds(o, 16)]
        py = pos_v[pl.ds(o + 128, 16)]
        h = ((hash_of(px) * PRIME_X) ^ (hash_of(py) * PRIME_Y)) \
            & (HASH_SIZE - 1)
        a_v[pl.ds(i * 16, 16)] = ((h >> 7) << 10) + (h & 127)

    def fire(j, g):
        return pltpu.async_copy(
            tab_hbm.at[idx_v.at[jnp.int32(j)]],
            out_t_v.at[jnp.int32(j // (_BPW // 128)),
                       pl.ds((j % (_BPW // 128)) * 128, 128)],
            gsems[g],
        )

    def build_and_fire(g):
        for d in range(8 * g, 8 * g + 8):
            c_d = (d >> 3) * (2 ** 23) + (d & 7) * 128
            for jj in range(_BPW // 128):
                for gg in range(8):
                    idx_v[d * (_BPW // 128) + jj, pl.ds(gg * 16, 16)] = (
                        a_v[pl.ds(jj * 128 + gg * 16, 16)] + c_d
                    )
                fire(d * (_BPW // 128) + jj, g)

    def drain_and_out(g):
        for _ in range(8 * (_BPW // 128)):
            pltpu.make_async_copy(
                tab_hbm.at[pl.ds(0, 128)],
                out_t_v.at[jnp.int32(0), pl.ds(0, 128)],
                gsems[g],
            ).wait()
        for gi in range(_BPW // 128):
            pltpu.async_copy(
                out_t_v.at[pl.ds(g * 8, 8), pl.ds(gi * 128, 128)],
                out4_hbm.at[jnp.int32(g), jnp.int32(wid * 4 + gi)],
                osem,
            )

    for g in range(DIM // 8):
        build_and_fire(g)
        if g >= 2:
            drain_and_out(g - 2)
    drain_and_out(DIM // 8 - 2)
    drain_and_out(DIM // 8 - 1)
    for _ in range((DIM // 8) * (_BPW // 128)):
        pltpu.make_async_copy(
            out_t_v.at[pl.ds(0, 8), pl.ds(0, 128)],
            out4_hbm.at[jnp.int32(0), jnp.int32(0)],
            osem,
        ).wait()


@jax.jit
def _hash_grid_lookup(pos_flat, tab_flat):
    mesh = plsc.VectorSubcoreMesh(core_axis_name="c", subcore_axis_name="s")
    k = functools.partial(
        pl.kernel,
        mesh=mesh,
        compiler_params=pltpu.CompilerParams(
            needs_layout_passes=False, use_tc_tiling_on_sc=False
        ),
        out_type=jax.ShapeDtypeStruct((DIM // 8, N // 128, 8, 128),
                                      jnp.float32),
        scratch_types=[
            pltpu.VMEM((2 * _BPW,), jnp.float32),
            pltpu.VMEM((_BPW,), jnp.int32),
            pltpu.VMEM((_NSTREAM, 128), jnp.int32),
            pltpu.VMEM((DIM, _BPW), jnp.float32),
        ] + [pltpu.SemaphoreType.DMA] * 9,
    )(_sc_body)
    return k(pos_flat, tab_flat)


def kernel(positions, table):
    pos_flat = (
        positions.reshape(N // 128, 128, 2).transpose(0, 2, 1).reshape(2 * N)
    )
    tab_flat = (
        table.reshape(HASH_SIZE // 128, 128, DIM // 8, 8)
        .transpose(2, 0, 3, 1)
        .reshape(HASH_SIZE * DIM)
    )
    out4 = _hash_grid_lookup(pos_flat, tab_flat)
    return out4.transpose(1, 3, 0, 2).reshape(N, DIM)
